# Initial kernel scaffold; baseline (speedup 1.0000x reference)
import jax
import jax.numpy as jnp
from jax import lax
from jax.experimental import pallas as pl
from jax.experimental.pallas import tpu as pltpu
from jax.experimental.pallas import tpu_sc as plsc

N = 10000
NPAD = 10240
E = 320000
NTILE = 16
ER = 158
EC = 128
ET = ER * EC
EPAD = NTILE * ET
DH = 64
ROWS_T = NPAD // NTILE
NQ = 8
NQW = NPAD // NQ


def _sc_body(xs, srcr, dstr, ewr, y, t0, t1, acc_sh, deg_sh,
             src_t, dst_t, nrm_t, lane_acc, dinv, bufa, bufb, zbuf, idb, sem):
    c = lax.axis_index("c")
    s = lax.axis_index("s")
    iota16 = lax.iota(jnp.int32, 16)
    zf16 = jnp.zeros((16,), jnp.float32)

    pltpu.sync_copy(srcr.at[s], src_t)
    pltpu.sync_copy(dstr.at[s], dst_t)
    pltpu.sync_copy(ewr.at[s], nrm_t)

    for j in range(5):
        for k in range(8):
            idb[j, pl.ds(16 * k, 16)] = iota16 + (128 * j + 16 * k)
    for k in range(40):
        zbuf[k, :] = zf16

    for q in range(NQ):
        lo = q * NQW

        @pl.loop(0, NQW, step=16)
        def _zero(o):
            for l in range(16):
                lane_acc[l, pl.ds(o, 16)] = zf16

        @pl.loop(0, ER)
        def _hist(r):
            for k in range(8):
                d = dst_t[r, pl.ds(16 * k, 16)]
                w = nrm_t[r, pl.ds(16 * k, 16)]
                dq = d - lo
                m = (dq >= 0) & (dq < NQW)
                dq = jnp.where(m, dq, 0)
                plsc.addupdate_scatter(lane_acc, [iota16, dq], w, mask=m)

        @pl.loop(0, NQW // 16)
        def _red(i):
            acc = lane_acc[0, pl.ds(16 * i, 16)]
            for l in range(1, 16):
                acc = acc + lane_acc[l, pl.ds(16 * i, 16)]
            dinv[lo // 16 + i, :] = acc

    pltpu.sync_copy(zbuf, deg_sh.at[pl.ds(s * 40, 40)])
    plsc.subcore_barrier()
    for j in range(5):
        pltpu.sync_copy(dinv.at[pl.ds(128 * j, 128)], deg_sh.at[idb.at[j]],
                        add=True)
    plsc.subcore_barrier()
    pltpu.sync_copy(deg_sh, dinv)

    @pl.loop(0, NPAD // 16)
    def _rsqrt(i):
        d = dinv[i, :] + 1.0
        yv = plsc.bitcast(
            jnp.int32(0x5F3759DF) - (plsc.bitcast(d, jnp.int32) >> 1),
            jnp.float32)
        for _ in range(3):
            yv = yv * (1.5 - 0.5 * d * yv * yv)
        dinv[i, :] = yv

    @pl.loop(0, ER)
    def _norm(r):
        for k in range(8):
            si = src_t[r, pl.ds(16 * k, 16)]
            di = dst_t[r, pl.ds(16 * k, 16)]
            w = nrm_t[r, pl.ds(16 * k, 16)]
            a = plsc.load_gather(dinv, [si >> 4, si & 15])
            b = plsc.load_gather(dinv, [di >> 4, di & 15])
            nrm_t[r, pl.ds(16 * k, 16)] = a * w * b

    base = s * ROWS_T

    def do_round(h_in, h_out):
        for b in range(ROWS_T // EC):
            rb = base + EC * b
            pltpu.async_copy(h_in.at[pl.ds(rb, EC)], bufa, sem).wait()

            @pl.loop(0, EC)
            def _self(e):
                g = rb + e
                dv = dinv[g >> 4, g & 15]
                sw = dv * dv
                for j in range(4):
                    bufa[e, pl.ds(16 * j, 16)] = bufa[e, pl.ds(16 * j, 16)] * sw

            pltpu.sync_copy(bufa, acc_sh.at[pl.ds(rb, EC)])
        plsc.subcore_barrier()

        @pl.loop(0, ER)
        def _edges(r):
            pltpu.async_copy(h_in.at[src_t.at[r]], bufb, sem).wait()

            @pl.loop(0, EC)
            def _scale(e):
                nv = nrm_t[r, e]
                for j in range(4):
                    bufb[e, pl.ds(16 * j, 16)] = bufb[e, pl.ds(16 * j, 16)] * nv

            pltpu.sync_copy(bufb, acc_sh.at[dst_t.at[r]], add=True)
        plsc.subcore_barrier()

        for b in range(ROWS_T // EC):
            rb = base + EC * b
            pltpu.sync_copy(acc_sh.at[pl.ds(rb, EC)], bufa)
            pltpu.sync_copy(bufa, h_out.at[pl.ds(rb, EC)])
        plsc.subcore_barrier()

    do_round(xs.at[c], t0.at[c])
    do_round(t0.at[c], t1.at[c])
    do_round(t1.at[c], y.at[c])


_sc_call = pl.kernel(
    _sc_body,
    out_type=[jax.ShapeDtypeStruct((2, NPAD, DH), jnp.float32)] * 3,
    mesh=plsc.VectorSubcoreMesh(core_axis_name="c", subcore_axis_name="s"),
    scratch_types=[
        pltpu.VMEM_SHARED((NPAD, DH), jnp.float32),
        pltpu.VMEM_SHARED((NPAD // 16, 16), jnp.float32),
        pltpu.VMEM((ER, EC), jnp.int32),
        pltpu.VMEM((ER, EC), jnp.int32),
        pltpu.VMEM((ER, EC), jnp.float32),
        pltpu.VMEM((16, NQW), jnp.float32),
        pltpu.VMEM((NPAD // 16, 16), jnp.float32),
        pltpu.VMEM((EC, DH), jnp.float32),
        pltpu.VMEM((EC, DH), jnp.float32),
        pltpu.VMEM((40, 16), jnp.float32),
        pltpu.VMEM((5, EC), jnp.int32),
        pltpu.SemaphoreType.DMA,
    ],
    name="gconv_prop_sc",
)


def _tc_body(y_ref, wi_ref, wc_ref, wo_ref, p_ref, hn_ref, cn_ref):
    yv = y_ref[...]
    P = p_ref[...]
    b_i, b_c, b_o, w_co = P[0], P[1], P[2], P[3]
    g_h, bt_h, g_c, bt_c = P[4], P[5], P[6], P[7]
    gi = jax.nn.sigmoid(
        jnp.dot(yv, wi_ref[...], preferred_element_type=jnp.float32) + b_i)
    gt = jnp.tanh(
        jnp.dot(yv, wc_ref[...], preferred_element_type=jnp.float32) + b_c)
    cn = gi * gt
    go = jax.nn.sigmoid(
        jnp.dot(yv, wo_ref[...], preferred_element_type=jnp.float32)
        + w_co * cn + b_o)
    hn = go * jnp.tanh(cn)

    def ln(v, g, b):
        mu = jnp.mean(v, axis=-1, keepdims=True)
        var = jnp.mean((v - mu) * (v - mu), axis=-1, keepdims=True)
        return (v - mu) * lax.rsqrt(var + 1e-5) * g + b

    hn_ref[...] = ln(hn, g_h, bt_h)
    cn_ref[...] = ln(cn, g_c, bt_c)


_BLK = 1024
_tc_call = pl.pallas_call(
    _tc_body,
    grid=(NPAD // _BLK,),
    in_specs=[
        pl.BlockSpec((_BLK, 128), lambda i: (i, 0)),
        pl.BlockSpec((128, 128), lambda i: (0, 0)),
        pl.BlockSpec((128, 128), lambda i: (0, 0)),
        pl.BlockSpec((128, 128), lambda i: (0, 0)),
        pl.BlockSpec((8, 128), lambda i: (0, 0)),
    ],
    out_specs=[
        pl.BlockSpec((_BLK, 128), lambda i: (i, 0)),
        pl.BlockSpec((_BLK, 128), lambda i: (i, 0)),
    ],
    out_shape=[jax.ShapeDtypeStruct((NPAD, 128), jnp.float32)] * 2,
)


def kernel(X, edge_index, edge_weight, W_xi, W_hi, W_xf, W_hf, W_xc, W_hc,
           W_xo, W_ho, b_i, b_f, b_c, b_o, w_ci, w_cf, w_co, g_h, bt_h,
           g_c, bt_c):
    x = X[0]
    xp = jnp.zeros((NPAD, 128), jnp.float32).at[:N].set(x)
    xs = jnp.stack([xp[:, :DH], xp[:, DH:]])
    src = jnp.pad(edge_index[0], (0, EPAD - E)).reshape(NTILE, ER, EC)
    dst = jnp.pad(edge_index[1], (0, EPAD - E)).reshape(NTILE, ER, EC)
    ew = jnp.pad(edge_weight, (0, EPAD - E)).reshape(NTILE, ER, EC)
    y2, _, _ = _sc_call(xs, src, dst, ew)
    Y = jnp.concatenate([y2[0], y2[1]], axis=1)
    P = jnp.stack([b_i, b_c, b_o, w_co, g_h, bt_h, g_c, bt_c])
    Hn, Cn = _tc_call(Y, W_xi, W_xc, W_xo, P)
    return Hn[None, :N], Cn[None, :N]


# speedup vs baseline: 17.4944x; 17.4944x over previous
import jax
import jax.numpy as jnp
from jax import lax
from jax.experimental import pallas as pl
from jax.experimental.pallas import tpu as pltpu
from jax.experimental.pallas import tpu_sc as plsc

N = 10000
NPAD = 10240
E = 320000
NTILE = 16
ER = 158
EC = 128
ET = ER * EC
EPAD = NTILE * ET
DH = 64
ROWS_T = NPAD // NTILE
NQ = 32
NQW = NPAD // NQ
DEGR = NPAD // DH


def _sc_body(xs, pkr, ewr, y, t0, t1, acc_sh, deg_sh,
             pk_t, nrm_t, lane_acc, dinv, deg2, bufa, bufb, sidxb, didxb,
             zbuf, idb, sem):
    c = lax.axis_index("c")
    s = lax.axis_index("s")
    iota16 = lax.iota(jnp.int32, 16)
    zf16 = jnp.zeros((16,), jnp.float32)

    pltpu.sync_copy(pkr.at[s], pk_t)
    pltpu.sync_copy(ewr.at[s], nrm_t)

    for j in range(2):
        for k in range(5):
            idb[j, pl.ds(16 * k, 16)] = iota16 + (80 * j + 16 * k)
    for k in range(10):
        for k2 in range(4):
            zbuf[k, pl.ds(16 * k2, 16)] = zf16

    for q in range(NQ):
        lo = q * NQW

        @pl.loop(0, 16 * NQW, step=16)
        def _zero(o):
            lane_acc[pl.ds(o, 16)] = zf16

        lane_base = iota16 * NQW

        @pl.loop(0, ER)
        def _hist(r):
            for k in range(8):
                p = pk_t[r, pl.ds(16 * k, 16)]
                w = nrm_t[r, pl.ds(16 * k, 16)]
                dq = (p & 16383) - lo
                m = (dq >= 0) & (dq < NQW)
                dq = jnp.where(m, dq, 0)
                plsc.addupdate_scatter(lane_acc, [lane_base + dq], w, mask=m)

        @pl.loop(0, NQW // 16)
        def _red(i):
            acc = lane_acc[pl.ds(16 * i, 16)]
            for l in range(1, 16):
                acc = acc + lane_acc[pl.ds(l * NQW + 16 * i, 16)]
            deg2[NQW // DH * q + (i >> 2), pl.ds((i & 3) * 16, 16)] = acc

    pltpu.sync_copy(zbuf, deg_sh.at[pl.ds(s * (DEGR // NTILE), DEGR // NTILE)])
    plsc.subcore_barrier()
    for j in range(2):
        pltpu.sync_copy(deg2.at[pl.ds(80 * j, 80)], deg_sh.at[idb.at[j]],
                        add=True)
    plsc.subcore_barrier()
    pltpu.sync_copy(deg_sh, deg2)

    @pl.loop(0, DEGR)
    def _rsqrt(i):
        for k in range(4):
            d = deg2[i, pl.ds(16 * k, 16)] + 1.0
            yv = plsc.bitcast(
                jnp.int32(0x5F3759DF) - (plsc.bitcast(d, jnp.int32) >> 1),
                jnp.float32)
            for _ in range(3):
                yv = yv * (1.5 - 0.5 * d * yv * yv)
            dinv[pl.ds(DH * i + 16 * k, 16)] = yv

    @pl.loop(0, ER)
    def _norm(r):
        for k in range(8):
            p = pk_t[r, pl.ds(16 * k, 16)]
            w = nrm_t[r, pl.ds(16 * k, 16)]
            a = plsc.load_gather(dinv, [p >> 14])
            b = plsc.load_gather(dinv, [p & 16383])
            nrm_t[r, pl.ds(16 * k, 16)] = a * w * b

    base = s * ROWS_T

    def do_round(h_in, h_out):
        for b in range(ROWS_T // EC):
            rb = base + EC * b
            pltpu.async_copy(h_in.at[pl.ds(rb, EC)], bufa, sem).wait()

            @pl.loop(0, EC // 16)
            def _self(eb):
                dv16 = dinv[pl.ds(rb + 16 * eb, 16)]
                sw16 = dv16 * dv16
                for l in range(16):
                    sw = sw16[l]
                    for j in range(4):
                        e = 16 * eb + l
                        bufa[e, pl.ds(16 * j, 16)] = (
                            bufa[e, pl.ds(16 * j, 16)] * sw)

            pltpu.sync_copy(bufa, acc_sh.at[pl.ds(rb, EC)])
        plsc.subcore_barrier()

        @pl.loop(0, ER)
        def _edges(r):
            for k in range(8):
                p = pk_t[r, pl.ds(16 * k, 16)]
                sidxb[pl.ds(16 * k, 16)] = p >> 14
                didxb[pl.ds(16 * k, 16)] = p & 16383
            pltpu.async_copy(h_in.at[sidxb], bufb, sem).wait()

            @pl.loop(0, EC // 16)
            def _scale(eb):
                nv16 = nrm_t[r, pl.ds(16 * eb, 16)]
                for l in range(16):
                    nv = nv16[l]
                    for j in range(4):
                        e = 16 * eb + l
                        bufb[e, pl.ds(16 * j, 16)] = (
                            bufb[e, pl.ds(16 * j, 16)] * nv)

            pltpu.sync_copy(bufb, acc_sh.at[didxb], add=True)
        plsc.subcore_barrier()

        for b in range(ROWS_T // EC):
            rb = base + EC * b
            pltpu.sync_copy(acc_sh.at[pl.ds(rb, EC)], bufa)
            pltpu.sync_copy(bufa, h_out.at[pl.ds(rb, EC)])
        plsc.subcore_barrier()

    do_round(xs.at[c], t0.at[c])
    do_round(t0.at[c], t1.at[c])
    do_round(t1.at[c], y.at[c])


_sc_call = pl.kernel(
    _sc_body,
    out_type=[jax.ShapeDtypeStruct((2, NPAD, DH), jnp.float32)] * 3,
    mesh=plsc.VectorSubcoreMesh(core_axis_name="c", subcore_axis_name="s"),
    scratch_types=[
        pltpu.VMEM_SHARED((NPAD, DH), jnp.float32),
        pltpu.VMEM_SHARED((DEGR, DH), jnp.float32),
        pltpu.VMEM((ER, EC), jnp.int32),
        pltpu.VMEM((ER, EC), jnp.float32),
        pltpu.VMEM((16 * NQW,), jnp.float32),
        pltpu.VMEM((NPAD,), jnp.float32),
        pltpu.VMEM((DEGR, DH), jnp.float32),
        pltpu.VMEM((EC, DH), jnp.float32),
        pltpu.VMEM((EC, DH), jnp.float32),
        pltpu.VMEM((EC,), jnp.int32),
        pltpu.VMEM((EC,), jnp.int32),
        pltpu.VMEM((DEGR // NTILE, DH), jnp.float32),
        pltpu.VMEM((2, 80), jnp.int32),
        pltpu.SemaphoreType.DMA,
    ],
    compiler_params=pltpu.CompilerParams(needs_layout_passes=False,
                                         use_tc_tiling_on_sc=False),
    name="gconv_prop_sc",
)


def _tc_body(y_ref, wi_ref, wc_ref, wo_ref, p_ref, hn_ref, cn_ref):
    yv = y_ref[...]
    P = p_ref[...]
    b_i, b_c, b_o, w_co = P[0], P[1], P[2], P[3]
    g_h, bt_h, g_c, bt_c = P[4], P[5], P[6], P[7]
    gi = jax.nn.sigmoid(
        jnp.dot(yv, wi_ref[...], preferred_element_type=jnp.float32) + b_i)
    gt = jnp.tanh(
        jnp.dot(yv, wc_ref[...], preferred_element_type=jnp.float32) + b_c)
    cn = gi * gt
    go = jax.nn.sigmoid(
        jnp.dot(yv, wo_ref[...], preferred_element_type=jnp.float32)
        + w_co * cn + b_o)
    hn = go * jnp.tanh(cn)

    def ln(v, g, b):
        mu = jnp.mean(v, axis=-1, keepdims=True)
        var = jnp.mean((v - mu) * (v - mu), axis=-1, keepdims=True)
        return (v - mu) * lax.rsqrt(var + 1e-5) * g + b

    hn_ref[...] = ln(hn, g_h, bt_h)
    cn_ref[...] = ln(cn, g_c, bt_c)


_BLK = 1024
_tc_call = pl.pallas_call(
    _tc_body,
    grid=(NPAD // _BLK,),
    in_specs=[
        pl.BlockSpec((_BLK, 128), lambda i: (i, 0)),
        pl.BlockSpec((128, 128), lambda i: (0, 0)),
        pl.BlockSpec((128, 128), lambda i: (0, 0)),
        pl.BlockSpec((128, 128), lambda i: (0, 0)),
        pl.BlockSpec((8, 128), lambda i: (0, 0)),
    ],
    out_specs=[
        pl.BlockSpec((_BLK, 128), lambda i: (i, 0)),
        pl.BlockSpec((_BLK, 128), lambda i: (i, 0)),
    ],
    out_shape=[jax.ShapeDtypeStruct((NPAD, 128), jnp.float32)] * 2,
)


def kernel(X, edge_index, edge_weight, W_xi, W_hi, W_xf, W_hf, W_xc, W_hc,
           W_xo, W_ho, b_i, b_f, b_c, b_o, w_ci, w_cf, w_co, g_h, bt_h,
           g_c, bt_c):
    x = X[0]
    xp = jnp.zeros((NPAD, 128), jnp.float32).at[:N].set(x)
    xs = jnp.stack([xp[:, :DH], xp[:, DH:]])
    pk = (edge_index[0] << 14) | edge_index[1]
    pk = jnp.pad(pk, (0, EPAD - E)).reshape(NTILE, ER, EC)
    ew = jnp.pad(edge_weight, (0, EPAD - E)).reshape(NTILE, ER, EC)
    y2, _, _ = _sc_call(xs, pk, ew)
    Y = jnp.concatenate([y2[0], y2[1]], axis=1)
    P = jnp.stack([b_i, b_c, b_o, w_co, g_h, bt_h, g_c, bt_c])
    Hn, Cn = _tc_call(Y, W_xi, W_xc, W_xo, P)
    return Hn[None, :N], Cn[None, :N]
